# baseline (speedup 1.0000x reference)
import functools

import numpy as np
import jax
import jax.numpy as jnp
from jax.experimental import pallas as pl
from jax.experimental.pallas import tpu as pltpu

_SIZES = [48, 24, 12, 6, 3]
_STRIDES = [8, 16, 32, 64, 128]
_C = 192
_NCLS = 80
_B = 2
_EPS = 1e-5
_PREC = jax.lax.Precision.HIGHEST


def _mm(a, b):
    return jnp.dot(a, b, precision=_PREC, preferred_element_type=jnp.float32)


def _level_body(Wp, E, CB, T, stride,
                x_ref, w1_ref, b1_ref, w2c_ref, b2c_ref, w2r_ref, b2r_ref,
                wfc_ref, wfr_ref, bf_ref, mask_ref, smask_ref,
                out_ref, h1_ref):
    i = pl.program_id(0)

    @pl.when(i == 0)
    def _init():
        z = jnp.zeros((2 * _C, E), jnp.float32)
        h1_ref[0, :, pl.ds(0, E)] = z
        h1_ref[T - 1, :, pl.ds(CB + E, E)] = z

    @pl.when(i < T)
    def _conv1():
        acc = None
        for dy in range(3):
            for dx in range(3):
                off = dy * Wp + dx
                t = _mm(w1_ref[3 * dy + dx], x_ref[0, :, off:off + CB])
                acc = t if acc is None else acc + t
        h1 = jnp.maximum(acc + b1_ref[:, :], 0.0) * mask_ref[0]
        h1_ref[pl.ds(i, 1), :, pl.ds(E, CB)] = h1[None]

        @pl.when(i > 0)
        def _():
            h1_ref[pl.ds(i - 1, 1), :, pl.ds(CB + E, E)] = h1[None, :, 0:E]

        @pl.when(i < T - 1)
        def _():
            h1_ref[pl.ds(i + 1, 1), :, pl.ds(0, E)] = h1[None, :, CB - E:CB]

    @pl.when(i >= 1)
    def _conv2():
        j = i - 1
        acc_c = None
        acc_r = None
        for dy in range(3):
            for dx in range(3):
                off = dy * Wp + dx
                tc = _mm(w2c_ref[3 * dy + dx],
                         h1_ref[pl.ds(j, 1), 0:_C, pl.ds(off, CB)][0])
                tr = _mm(w2r_ref[3 * dy + dx],
                         h1_ref[pl.ds(j, 1), _C:2 * _C, pl.ds(off, CB)][0])
                acc_c = tc if acc_c is None else acc_c + tc
                acc_r = tr if acc_r is None else acc_r + tr
        h2c = jnp.maximum(acc_c + b2c_ref[:, :], 0.0)
        h2r = jnp.maximum(acc_r + b2r_ref[:, :], 0.0)
        yf = _mm(wfc_ref[...], h2c) + _mm(wfr_ref[...], h2r) + bf_ref[:, :]
        out_ref[0] = jnp.where(smask_ref[:, :] > 0.0,
                               jnp.maximum(yf * stride, 0.0), yf)


def _fold_bn(w, b, g, be, mu, va):
    scale = g / jnp.sqrt(va + _EPS)
    wf = w * scale[:, None, None, None]
    bf = (b - mu) * scale + be
    return wf, bf


def _taps(w):
    return jnp.transpose(w, (2, 3, 0, 1)).reshape(9, w.shape[0], w.shape[1])


def _full_spec(shape):
    nd = len(shape)
    return pl.BlockSpec(shape, lambda i: (0,) * nd)


def kernel(fpn0, fpn1, fpn2, fpn3, fpn4,
           cls_convs_w, cls_convs_b, cls_bn_gamma, cls_bn_beta, cls_bn_mean,
           cls_bn_var, cls_final_w, cls_final_b,
           reg_convs_w, reg_convs_b, reg_bn_gamma, reg_bn_beta, reg_bn_mean,
           reg_bn_var, reg_final_w, reg_final_b):
    fpns = (fpn0, fpn1, fpn2, fpn3, fpn4)

    w1c, b1c = _fold_bn(cls_convs_w[0], cls_convs_b[0], cls_bn_gamma[0],
                        cls_bn_beta[0], cls_bn_mean[0], cls_bn_var[0])
    w1r, b1r = _fold_bn(reg_convs_w[0], reg_convs_b[0], reg_bn_gamma[0],
                        reg_bn_beta[0], reg_bn_mean[0], reg_bn_var[0])
    w2c, b2c = _fold_bn(cls_convs_w[1], cls_convs_b[1], cls_bn_gamma[1],
                        cls_bn_beta[1], cls_bn_mean[1], cls_bn_var[1])
    w2r, b2r = _fold_bn(reg_convs_w[1], reg_convs_b[1], reg_bn_gamma[1],
                        reg_bn_beta[1], reg_bn_mean[1], reg_bn_var[1])

    w1 = _taps(jnp.concatenate([w1c, w1r], axis=0))
    b1 = jnp.concatenate([b1c, b1r])[:, None]
    w2c_t = _taps(w2c)
    w2r_t = _taps(w2r)
    b2c_v = b2c[:, None]
    b2r_v = b2r[:, None]

    wfc = jnp.concatenate([cls_final_w[:, :, 0, 0],
                           jnp.zeros((5, _C), jnp.float32)], axis=0)
    wfr = jnp.concatenate([jnp.zeros((_NCLS, _C), jnp.float32),
                           reg_final_w[:, :, 0, 0]], axis=0)
    bf = jnp.concatenate([cls_final_b, reg_final_b])[:, None]
    smask = jnp.asarray(
        np.concatenate([np.zeros(81, np.float32),
                        np.ones(4, np.float32)])[:, None])

    outs_cls, outs_reg, outs_cent = [], [], []
    for lvl, (x, H, stride) in enumerate(zip(fpns, _SIZES, _STRIDES)):
        Hp = H + 2
        Wp = H + 2
        P = _B * Hp * Wp
        E = Wp + 1
        CB = min(512, -(-P // 128) * 128)
        T = -(-P // CB)
        Ppad = T * CB

        xpad = jnp.pad(x, ((0, 0), (0, 0), (1, 1), (1, 1)))
        x_cat = jnp.transpose(xpad, (1, 0, 2, 3)).reshape(_C, P)
        x_full = jnp.pad(x_cat, ((0, 0), (E, E + Ppad - P)))
        x_ov = jnp.stack([x_full[:, j * CB:j * CB + CB + 2 * E]
                          for j in range(T)])

        m = np.zeros((_B, Hp, Wp), np.float32)
        m[:, 1:H + 1, 1:H + 1] = 1.0
        m = np.pad(m.reshape(P), (0, Ppad - P)).reshape(T, 1, CB)
        mask_ov = jnp.asarray(m)

        body = functools.partial(_level_body, Wp, E, CB, T, float(stride))
        out = pl.pallas_call(
            body,
            grid=(T + 1,),
            in_specs=[
                pl.BlockSpec((1, _C, CB + 2 * E),
                             lambda i, _T=T: (jnp.minimum(i, _T - 1), 0, 0)),
                _full_spec(w1.shape),
                _full_spec(b1.shape),
                _full_spec(w2c_t.shape),
                _full_spec(b2c_v.shape),
                _full_spec(w2r_t.shape),
                _full_spec(b2r_v.shape),
                _full_spec(wfc.shape),
                _full_spec(wfr.shape),
                _full_spec(bf.shape),
                pl.BlockSpec((1, 1, CB),
                             lambda i, _T=T: (jnp.minimum(i, _T - 1), 0, 0)),
                _full_spec(smask.shape),
            ],
            out_specs=pl.BlockSpec((1, 85, CB),
                                   lambda i: (jnp.maximum(i - 1, 0), 0, 0)),
            out_shape=jax.ShapeDtypeStruct((T, 85, CB), jnp.float32),
            scratch_shapes=[
                pltpu.VMEM((T, 2 * _C, CB + 2 * E), jnp.float32),
            ],
        )(x_ov, w1, b1, w2c_t, b2c_v, w2r_t, b2r_v, wfc, wfr, bf,
          mask_ov, smask)

        o = jnp.transpose(out, (1, 0, 2)).reshape(85, Ppad)[:, :P]
        o = jnp.transpose(o.reshape(85, _B, Hp, Wp), (1, 0, 2, 3))
        o = o[:, :, 1:H + 1, 1:H + 1]
        outs_cls.append(o[:, 0:_NCLS])
        outs_cent.append(o[:, _NCLS:_NCLS + 1])
        outs_reg.append(o[:, _NCLS + 1:_NCLS + 5])

    return tuple(outs_cls) + tuple(outs_reg) + tuple(outs_cent)


# speedup vs baseline: 2.5340x; 2.5340x over previous
import functools

import numpy as np
import jax
import jax.numpy as jnp
from jax.experimental import pallas as pl
from jax.experimental.pallas import tpu as pltpu

_SIZES = [48, 24, 12, 6, 3]
_STRIDES = [8, 16, 32, 64, 128]
_C = 192
_NCLS = 80
_B = 2
_EPS = 1e-5


def _mm(a, b):
    return jnp.dot(a, b, preferred_element_type=jnp.float32)


def _level_body(Wp, E, CB, T, stride,
                x_ref, w1_ref, a1_ref, w2c_ref, a2c_ref, w2r_ref, a2r_ref,
                wfc_ref, wfr_ref, bf_ref, mask_ref, smask_ref,
                out_ref, h1_ref):
    i = pl.program_id(0)

    @pl.when(i == 0)
    def _init():
        z = jnp.zeros((2 * _C, E), jnp.bfloat16)
        h1_ref[0, :, pl.ds(0, E)] = z
        h1_ref[T - 1, :, pl.ds(CB + E, E)] = z

    @pl.when(i < T)
    def _conv1():
        acc = None
        for dy in range(3):
            for dx in range(3):
                off = dy * Wp + dx
                t = _mm(w1_ref[3 * dy + dx], x_ref[0, :, off:off + CB])
                acc = t if acc is None else acc + t
        h1 = jnp.maximum(acc * a1_ref[:, 0:1] + a1_ref[:, 1:2], 0.0)
        h1 = (h1 * mask_ref[0]).astype(jnp.bfloat16)
        h1_ref[pl.ds(i, 1), :, pl.ds(E, CB)] = h1[None]

        @pl.when(i > 0)
        def _():
            h1_ref[pl.ds(i - 1, 1), :, pl.ds(CB + E, E)] = h1[None, :, 0:E]

        @pl.when(i < T - 1)
        def _():
            h1_ref[pl.ds(i + 1, 1), :, pl.ds(0, E)] = h1[None, :, CB - E:CB]

    @pl.when(i >= 1)
    def _conv2():
        j = i - 1
        acc_c = None
        acc_r = None
        for dy in range(3):
            for dx in range(3):
                off = dy * Wp + dx
                tc = _mm(w2c_ref[3 * dy + dx],
                         h1_ref[pl.ds(j, 1), 0:_C, pl.ds(off, CB)][0])
                tr = _mm(w2r_ref[3 * dy + dx],
                         h1_ref[pl.ds(j, 1), _C:2 * _C, pl.ds(off, CB)][0])
                acc_c = tc if acc_c is None else acc_c + tc
                acc_r = tr if acc_r is None else acc_r + tr
        h2c = jnp.maximum(acc_c * a2c_ref[:, 0:1] + a2c_ref[:, 1:2], 0.0)
        h2r = jnp.maximum(acc_r * a2r_ref[:, 0:1] + a2r_ref[:, 1:2], 0.0)
        yf = (_mm(wfc_ref[...], h2c.astype(jnp.bfloat16))
              + _mm(wfr_ref[...], h2r.astype(jnp.bfloat16))
              + bf_ref[:, :])
        out_ref[0] = jnp.where(smask_ref[:, :] > 0.0,
                               jnp.maximum(yf * stride, 0.0), yf)


def _bn_affine(b, g, be, mu, va):
    scale = g / jnp.sqrt(va + _EPS)
    shift = (b - mu) * scale + be
    return jnp.stack([scale, shift], axis=1)


def _taps(w):
    t = jnp.transpose(w, (2, 3, 0, 1)).reshape(9, w.shape[0], w.shape[1])
    return t.astype(jnp.bfloat16)


def _full_spec(shape):
    nd = len(shape)
    return pl.BlockSpec(shape, lambda i: (0,) * nd)


def kernel(fpn0, fpn1, fpn2, fpn3, fpn4,
           cls_convs_w, cls_convs_b, cls_bn_gamma, cls_bn_beta, cls_bn_mean,
           cls_bn_var, cls_final_w, cls_final_b,
           reg_convs_w, reg_convs_b, reg_bn_gamma, reg_bn_beta, reg_bn_mean,
           reg_bn_var, reg_final_w, reg_final_b):
    fpns = (fpn0, fpn1, fpn2, fpn3, fpn4)

    w1 = _taps(jnp.concatenate([cls_convs_w[0], reg_convs_w[0]], axis=0))
    a1 = jnp.concatenate([
        _bn_affine(cls_convs_b[0], cls_bn_gamma[0], cls_bn_beta[0],
                   cls_bn_mean[0], cls_bn_var[0]),
        _bn_affine(reg_convs_b[0], reg_bn_gamma[0], reg_bn_beta[0],
                   reg_bn_mean[0], reg_bn_var[0])], axis=0)
    w2c_t = _taps(cls_convs_w[1])
    w2r_t = _taps(reg_convs_w[1])
    a2c = _bn_affine(cls_convs_b[1], cls_bn_gamma[1], cls_bn_beta[1],
                     cls_bn_mean[1], cls_bn_var[1])
    a2r = _bn_affine(reg_convs_b[1], reg_bn_gamma[1], reg_bn_beta[1],
                     reg_bn_mean[1], reg_bn_var[1])

    wfc = jnp.concatenate([cls_final_w[:, :, 0, 0],
                           jnp.zeros((5, _C), jnp.float32)],
                          axis=0).astype(jnp.bfloat16)
    wfr = jnp.concatenate([jnp.zeros((_NCLS, _C), jnp.float32),
                           reg_final_w[:, :, 0, 0]],
                          axis=0).astype(jnp.bfloat16)
    bf = jnp.concatenate([cls_final_b, reg_final_b])[:, None]
    smask = jnp.asarray(
        np.concatenate([np.zeros(81, np.float32),
                        np.ones(4, np.float32)])[:, None])

    outs_cls, outs_reg, outs_cent = [], [], []
    for lvl, (x, H, stride) in enumerate(zip(fpns, _SIZES, _STRIDES)):
        Hp = H + 2
        Wp = H + 2
        P = _B * Hp * Wp
        E = Wp + 1
        CB = min(512, -(-P // 128) * 128)
        T = -(-P // CB)
        Ppad = T * CB

        xpad = jnp.pad(x, ((0, 0), (0, 0), (1, 1), (1, 1)))
        x_cat = jnp.transpose(xpad, (1, 0, 2, 3)).reshape(_C, P)
        x_full = jnp.pad(x_cat, ((0, 0), (E, E + Ppad - P)))
        x_ov = jnp.stack([x_full[:, j * CB:j * CB + CB + 2 * E]
                          for j in range(T)]).astype(jnp.bfloat16)

        m = np.zeros((_B, Hp, Wp), np.float32)
        m[:, 1:H + 1, 1:H + 1] = 1.0
        m = np.pad(m.reshape(P), (0, Ppad - P)).reshape(T, 1, CB)
        mask_ov = jnp.asarray(m)

        body = functools.partial(_level_body, Wp, E, CB, T, float(stride))
        out = pl.pallas_call(
            body,
            grid=(T + 1,),
            in_specs=[
                pl.BlockSpec((1, _C, CB + 2 * E),
                             lambda i, _T=T: (jnp.minimum(i, _T - 1), 0, 0)),
                _full_spec(w1.shape),
                _full_spec(a1.shape),
                _full_spec(w2c_t.shape),
                _full_spec(a2c.shape),
                _full_spec(w2r_t.shape),
                _full_spec(a2r.shape),
                _full_spec(wfc.shape),
                _full_spec(wfr.shape),
                _full_spec(bf.shape),
                pl.BlockSpec((1, 1, CB),
                             lambda i, _T=T: (jnp.minimum(i, _T - 1), 0, 0)),
                _full_spec(smask.shape),
            ],
            out_specs=pl.BlockSpec((1, 85, CB),
                                   lambda i: (jnp.maximum(i - 1, 0), 0, 0)),
            out_shape=jax.ShapeDtypeStruct((T, 85, CB), jnp.float32),
            scratch_shapes=[
                pltpu.VMEM((T, 2 * _C, CB + 2 * E), jnp.bfloat16),
            ],
        )(x_ov, w1, a1, w2c_t, a2c, w2r_t, a2r, wfc, wfr, bf,
          mask_ov, smask)

        o = jnp.transpose(out, (1, 0, 2)).reshape(85, Ppad)[:, :P]
        o = jnp.transpose(o.reshape(85, _B, Hp, Wp), (1, 0, 2, 3))
        o = o[:, :, 1:H + 1, 1:H + 1]
        outs_cls.append(o[:, 0:_NCLS])
        outs_cent.append(o[:, _NCLS:_NCLS + 1])
        outs_reg.append(o[:, _NCLS + 1:_NCLS + 5])

    return tuple(outs_cls) + tuple(outs_reg) + tuple(outs_cent)
